# Initial kernel scaffold; baseline (speedup 1.0000x reference)
import jax
import jax.numpy as jnp
from jax.experimental import pallas as pl


def kernel(x, c, image_features, gt, gt_normals, edge_index, Pid, params):
    raise NotImplementedError("write your pallas kernel here")


# speedup vs baseline: 2.5902x; 2.5902x over previous
import functools

import jax
import jax.numpy as jnp
from jax import lax
from jax.experimental import pallas as pl
from jax.experimental.pallas import tpu as pltpu
from jax.experimental.pallas import tpu_sc as plsc

N = 10000
N_PAD = 10240
E = 160000
FEAT = 128
DIM = 3
G = 2048
CH = 128
IMH = 56
IMW = 56
LAMBDA_N = 0.00016
LAMBDA_LAP = 0.3
LAMBDA_E = 0.3

NW = 32
NTILE = 16
E_TILE = E // NW
ECHUNK = 128
NCHUNK = 160 * 1024 // NW // ECHUNK
E_TILE_PAD = NCHUNK * ECHUNK
DUMMY = 10232
RPT = N_PAD // NTILE
GCHUNK = 64
GN_T = N_PAD // NW
GJ = GN_T // GCHUNK

@functools.lru_cache(maxsize=None)
def _mesh():
    return plsc.VectorSubcoreMesh(core_axis_name="c", subcore_axis_name="s",
                                  num_cores=2, num_subcores=NTILE)


def _wid():
    return lax.axis_index("s") * 2 + lax.axis_index("c")


def _zero_rows(rows, nrow, w):
    z = jnp.zeros((16,), jnp.float32)

    def body(r, _):
        for k in range(w // 16):
            rows[r, pl.ds(k * 16, 16)] = z
        return 0

    lax.fori_loop(0, nrow, body, 0)


SIZES_W128 = [5040] * 14 + [4800, 4640]
SIZES_W256 = [5040] * 11 + [4928] * 4 + [4848]


def _edge_layout(gather_ids, scatter_ids, half_sizes):
    sizes = jnp.asarray(half_sizes * 2, jnp.int32)
    starts = jnp.concatenate([
        jnp.cumsum(jnp.asarray([0] + half_sizes[:-1], jnp.int32)),
        E // 2 + jnp.cumsum(jnp.asarray([0] + half_sizes[:-1], jnp.int32))])
    pos = starts[:, None] + jnp.arange(E_TILE_PAD, dtype=jnp.int32)[None, :]
    valid = jnp.arange(E_TILE_PAD, dtype=jnp.int32)[None, :] < sizes[:, None]
    posc = jnp.clip(pos, 0, E - 1)
    g3 = jnp.where(valid, gather_ids[posc], 0).astype(jnp.int32)
    s3 = jnp.where(valid, scatter_ids[posc], DUMMY).astype(jnp.int32)
    return (g3.reshape(NW, NCHUNK, ECHUNK), s3.reshape(NW, NCHUNK, ECHUNK))


@functools.lru_cache(maxsize=None)
def _sc_segsum(w):
    def body(y_hbm, src_hbm, dst_hbm, out_hbm, idx_s, idx_d, rows, acc, sem):
        cid = lax.axis_index("c")
        sid = lax.axis_index("s")
        wid = _wid()
        _zero_rows(rows, ECHUNK, w)
        for k in range(RPT // ECHUNK):
            pltpu.sync_copy(rows, acc.at[pl.ds(sid * RPT + k * ECHUNK, ECHUNK)])
        pltpu.sync_copy(src_hbm.at[wid], idx_s)
        pltpu.sync_copy(dst_hbm.at[wid], idx_d)
        plsc.subcore_barrier()

        def step(j, _):
            pltpu.async_copy(y_hbm.at[idx_s.at[j]], rows, sem).wait()
            pltpu.sync_copy(rows, acc.at[idx_d.at[j]], add=True)
            return 0

        lax.fori_loop(0, NCHUNK, step, 0)
        plsc.subcore_barrier()
        for k in range(RPT // ECHUNK):
            pltpu.sync_copy(acc.at[pl.ds(sid * RPT + k * ECHUNK, ECHUNK)], rows)
            pltpu.sync_copy(rows, out_hbm.at[cid, pl.ds(sid * RPT + k * ECHUNK, ECHUNK)])

    return pl.kernel(
        body,
        out_type=jax.ShapeDtypeStruct((2, N_PAD, w), jnp.float32),
        mesh=_mesh(),
        compiler_params=pltpu.CompilerParams(use_tc_tiling_on_sc=(w == FEAT)),
        scratch_types=[
            pltpu.VMEM((NCHUNK, ECHUNK), jnp.int32),
            pltpu.VMEM((NCHUNK, ECHUNK), jnp.int32),
            pltpu.VMEM((ECHUNK, w), jnp.float32),
            pltpu.VMEM_SHARED((N_PAD, w), jnp.float32),
            pltpu.SemaphoreType.DMA,
        ],
    )


def _sc_segsum_exact_build():
    w = FEAT
    nsub = w // 16

    def body(y_hbm, eye_hbm, src_hbm, dst_hbm, out_hbm, idx_s, idx_d, rows,
             fbuf, eye, acc, sem):
        cid = lax.axis_index("c")
        sid = lax.axis_index("s")
        wid = _wid()
        _zero_rows(rows, ECHUNK, w)
        for k in range(RPT // ECHUNK):
            pltpu.sync_copy(rows, acc.at[pl.ds(sid * RPT + k * ECHUNK, ECHUNK)])
        pltpu.sync_copy(src_hbm.at[wid], idx_s)
        pltpu.sync_copy(dst_hbm.at[wid], idx_d)
        pltpu.sync_copy(eye_hbm, eye)
        plsc.subcore_barrier()

        dummy_vec = jnp.full((16,), DUMMY, jnp.int32)

        def chunk(j, carry):
            prev, cnt, fidx, acc8 = carry
            pltpu.async_copy(y_hbm.at[idx_s.at[j]], rows, sem).wait()

            def group(b, carry2):
                prev, cnt, fidx, acc8 = carry2
                dv = idx_d[j, pl.ds(b * 16, 16)]
                for k in range(16):
                    d = dv[k]
                    i = b * 16 + k
                    is_new = d != prev
                    fl = jnp.where(jnp.logical_and(is_new, prev >= 0), 1, 0)
                    drained = cnt >= 16

                    @pl.when(drained)
                    def _():
                        pltpu.sync_copy(fbuf, acc.at[fidx], add=True)

                    dr = jnp.where(drained, 1, 0)
                    cnt = cnt * (1 - dr)
                    fidx = fidx * (1 - dr) + DUMMY * dr

                    @pl.when(fl == 1)
                    def _():
                        for c in range(nsub):
                            fbuf[cnt, pl.ds(c * 16, 16)] = acc8[c]

                    m = eye[cnt, pl.ds(0, 16)] * fl
                    fidx = fidx * (1 - m) + prev * m
                    cnt = cnt + fl
                    reset = jnp.where(is_new, 0.0, 1.0)
                    acc8 = tuple(
                        acc8[c] * reset + rows[i, pl.ds(c * 16, 16)]
                        for c in range(nsub))
                    prev = d
                return prev, cnt, fidx, acc8

            return lax.fori_loop(0, ECHUNK // 16, group, (prev, cnt, fidx, acc8))

        prev, cnt, fidx, acc8 = lax.fori_loop(
            0, NCHUNK, chunk,
            (jnp.int32(-1), jnp.int32(0), dummy_vec,
             tuple(jnp.zeros((16,), jnp.float32) for _ in range(nsub))))

        drained = cnt >= 16

        @pl.when(drained)
        def _():
            pltpu.sync_copy(fbuf, acc.at[fidx], add=True)

        dr = jnp.where(drained, 1, 0)
        cnt = cnt * (1 - dr)
        fidx = fidx * (1 - dr) + DUMMY * dr
        fin = jnp.where(prev >= 0, 1, 0)

        @pl.when(fin == 1)
        def _():
            for c in range(nsub):
                fbuf[cnt, pl.ds(c * 16, 16)] = acc8[c]

        m = eye[cnt, pl.ds(0, 16)] * fin
        fidx = fidx * (1 - m) + prev * m
        pltpu.sync_copy(fbuf, acc.at[fidx], add=True)

        plsc.subcore_barrier()
        for k in range(RPT // ECHUNK):
            pltpu.sync_copy(acc.at[pl.ds(sid * RPT + k * ECHUNK, ECHUNK)], rows)
            pltpu.sync_copy(rows, out_hbm.at[cid, pl.ds(sid * RPT + k * ECHUNK, ECHUNK)])

    return pl.kernel(
        body,
        out_type=jax.ShapeDtypeStruct((2, N_PAD, w), jnp.float32),
        mesh=_mesh(),
        compiler_params=pltpu.CompilerParams(needs_layout_passes=False),
        scratch_types=[
            pltpu.VMEM((NCHUNK, ECHUNK), jnp.int32),
            pltpu.VMEM((NCHUNK, ECHUNK), jnp.int32),
            pltpu.VMEM((ECHUNK, w), jnp.float32),
            pltpu.VMEM((16, w), jnp.float32),
            pltpu.VMEM((16, 16), jnp.int32),
            pltpu.VMEM_SHARED((N_PAD, w), jnp.float32),
            pltpu.SemaphoreType.DMA,
        ],
    )


_SC_SEGSUM_EXACT = []


def _sc_segsum_exact():
    if not _SC_SEGSUM_EXACT:
        _SC_SEGSUM_EXACT.append(_sc_segsum_exact_build())
    return _SC_SEGSUM_EXACT[0]


@functools.lru_cache(maxsize=None)
def _sc_gather(t_rows, w):
    def body(table_hbm, idx_hbm, out_hbm, idxv, rows, sem):
        wid = _wid()
        pltpu.sync_copy(idx_hbm.at[wid], idxv)
        for j in range(GJ):
            pltpu.async_copy(table_hbm.at[idxv.at[j]], rows, sem).wait()
            pltpu.sync_copy(rows, out_hbm.at[pl.ds(wid * GN_T + j * GCHUNK, GCHUNK)])

    return pl.kernel(
        body,
        out_type=jax.ShapeDtypeStruct((N_PAD, w), jnp.float32),
        mesh=_mesh(),
        compiler_params=pltpu.CompilerParams(use_tc_tiling_on_sc=(w == FEAT)),
        scratch_types=[
            pltpu.VMEM((GJ, GCHUNK), jnp.int32),
            pltpu.VMEM((GCHUNK, w), jnp.float32),
            pltpu.SemaphoreType.DMA,
        ],
    )


def _rsqrt_nr(s):
    se = jnp.maximum(s, 1e-30)
    i = lax.bitcast_convert_type(se, jnp.int32)
    i = jnp.int32(0x5F3759DF) - lax.shift_right_arithmetic(i, 1)
    y = lax.bitcast_convert_type(i, jnp.float32)
    for _ in range(3):
        y = y * (1.5 - 0.5 * se * y * y)
    return y


def _sc_nloss_build():
    def body(c4_hbm, n4_hbm, src_hbm, dst_hbm, out_hbm, ct, nt, idx_s, idx_d,
             accv):
        wid = _wid()
        pltpu.sync_copy(c4_hbm, ct)
        pltpu.sync_copy(n4_hbm, nt)
        pltpu.sync_copy(src_hbm.at[wid], idx_s)
        pltpu.sync_copy(dst_hbm.at[wid], idx_d)

        def outer(j, acc):
            def inner(t, acc2):
                sv = idx_s[j, pl.ds(t * 16, 16)]
                dv = idx_d[j, pl.ds(t * 16, 16)]
                bs = sv * 4
                bd = dv * 4
                cxs = plsc.load_gather(ct, [bs])
                cys = plsc.load_gather(ct, [bs + 1])
                czs = plsc.load_gather(ct, [bs + 2])
                cxd = plsc.load_gather(ct, [bd])
                cyd = plsc.load_gather(ct, [bd + 1])
                czd = plsc.load_gather(ct, [bd + 2])
                nx = plsc.load_gather(nt, [bs])
                ny = plsc.load_gather(nt, [bs + 1])
                nz = plsc.load_gather(nt, [bs + 2])
                ex = cxd - cxs
                ey = cyd - cys
                ez = czd - czs
                se = ex * ex + ey * ey + ez * ez
                sn = nx * nx + ny * ny + nz * nz
                dot = ex * nx + ey * ny + ez * nz
                sqe = se * _rsqrt_nr(se)
                sqn = sn * _rsqrt_nr(sn)
                val = jnp.abs(dot) / ((sqe + 1e-12) * (sqn + 1e-12))
                val = jnp.where(dv < N, val, 0.0)
                return acc2 + val

            return lax.fori_loop(0, ECHUNK // 16, inner, acc)

        acc = lax.fori_loop(0, NCHUNK, outer, jnp.zeros((16,), jnp.float32))
        accv[pl.ds(0, 16)] = acc
        pltpu.sync_copy(accv, out_hbm.at[wid])

    return pl.kernel(
        body,
        out_type=jax.ShapeDtypeStruct((NW, 16), jnp.float32),
        mesh=_mesh(),
        compiler_params=pltpu.CompilerParams(needs_layout_passes=False),
        scratch_types=[
            pltpu.VMEM((N_PAD * 4,), jnp.float32),
            pltpu.VMEM((N_PAD * 4,), jnp.float32),
            pltpu.VMEM((NCHUNK, ECHUNK), jnp.int32),
            pltpu.VMEM((NCHUNK, ECHUNK), jnp.int32),
            pltpu.VMEM((16,), jnp.float32),
        ],
    )


MMB = 1024
CB = 512


def _row_spec(b, w):
    return pl.BlockSpec((b, w), lambda i: (i, 0))


def _full_spec(shape):
    nd = len(shape)
    return pl.BlockSpec(shape, lambda i: (0,) * nd)


def _parts_spec(b, w):
    return pl.BlockSpec((2, b, w), lambda i: (0, i, 0))


@functools.lru_cache(maxsize=None)
def _tc_gcn_in(interpret=False):
    def body(x, px, f, pf, d, w, b, o):
        deg1 = 1.0 + (d[0, :, 0] + d[1, :, 0])[:, None]
        u = jnp.concatenate([x[...] + px[0] + px[1],
                             f[...] + pf[0] + pf[1]], axis=1) / deg1
        o[...] = jnp.tanh(
            jnp.dot(u, w[...], preferred_element_type=jnp.float32) + b[...])

    return pl.pallas_call(
        body,
        grid=(N_PAD // MMB,),
        in_specs=[_row_spec(MMB, FEAT), _parts_spec(MMB, FEAT),
                  _row_spec(MMB, FEAT), _parts_spec(MMB, FEAT),
                  _parts_spec(MMB, 16), _full_spec((2 * FEAT, FEAT)),
                  _full_spec((1, FEAT))],
        out_specs=_row_spec(MMB, FEAT),
        out_shape=jax.ShapeDtypeStruct((N_PAD, FEAT), jnp.float32),
        interpret=interpret,
    )


@functools.lru_cache(maxsize=None)
def _tc_gcn_hidden(interpret=False):
    def body(h, p, d, w, b, o):
        deg1 = 1.0 + (d[0, :, 0] + d[1, :, 0])[:, None]
        u = (h[...] + p[0] + p[1]) / deg1
        o[...] = h[...] + jnp.tanh(
            jnp.dot(u, w[...], preferred_element_type=jnp.float32) + b[...])

    return pl.pallas_call(
        body,
        grid=(N_PAD // MMB,),
        in_specs=[_row_spec(MMB, FEAT), _parts_spec(MMB, FEAT),
                  _parts_spec(MMB, 16), _full_spec((FEAT, FEAT)),
                  _full_spec((1, FEAT))],
        out_specs=_row_spec(MMB, FEAT),
        out_shape=jax.ShapeDtypeStruct((N_PAD, FEAT), jnp.float32),
        interpret=interpret,
    )


@functools.lru_cache(maxsize=None)
def _tc_gcn_out(interpret=False):
    b = GN_T

    def body(h, p, d, w, b16, oc, opix):
        deg1 = 1.0 + (d[0, :, 0] + d[1, :, 0])[:, None]
        u = (h[...] + p[0] + p[1]) / deg1
        cnew = jnp.dot(u, w[...], preferred_element_type=jnp.float32) + b16[...]
        oc[...] = cnew
        px = jnp.clip(((cnew[:, 0] + 1.0) * 0.5 * (IMW - 1)).astype(jnp.int32),
                      0, IMW - 1)
        py = jnp.clip(((cnew[:, 1] + 1.0) * 0.5 * (IMH - 1)).astype(jnp.int32),
                      0, IMH - 1)
        opix[0, 0, :] = py * IMW + px

    return pl.pallas_call(
        body,
        grid=(N_PAD // b,),
        in_specs=[_row_spec(b, FEAT), _parts_spec(b, FEAT),
                  _parts_spec(b, 16), _full_spec((FEAT, 16)),
                  _full_spec((1, 16))],
        out_specs=[_row_spec(b, 16),
                   pl.BlockSpec((1, 1, b), lambda i: (i, 0, 0))],
        out_shape=[jax.ShapeDtypeStruct((N_PAD, 16), jnp.float32),
                   jax.ShapeDtypeStruct((NW, 1, GN_T), jnp.int32)],
        interpret=interpret,
    )


@functools.lru_cache(maxsize=None)
def _tc_pidx(interpret=False):
    b = GN_T

    def body(c16, opix):
        px = jnp.clip(((c16[:, 0] + 1.0) * 0.5 * (IMW - 1)).astype(jnp.int32),
                      0, IMW - 1)
        py = jnp.clip(((c16[:, 1] + 1.0) * 0.5 * (IMH - 1)).astype(jnp.int32),
                      0, IMH - 1)
        opix[0, 0, :] = py * IMW + px

    return pl.pallas_call(
        body,
        grid=(N_PAD // b,),
        in_specs=[_row_spec(b, 16)],
        out_specs=pl.BlockSpec((1, 1, b), lambda i: (i, 0, 0)),
        out_shape=jax.ShapeDtypeStruct((NW, 1, GN_T), jnp.int32),
        interpret=interpret,
    )


@functools.lru_cache(maxsize=None)
def _tc_chamfer(interpret=False):
    def body(c16, g16, s1, idx1, d2):
        i = pl.program_id(0)
        cb = c16[...]
        gb = g16[...]
        c2 = jnp.sum(cb * cb, axis=1, keepdims=True)
        g2 = jnp.sum(gb * gb, axis=1)[None, :]
        d = c2 + g2 - 2.0 * lax.dot_general(
            cb, gb, (((1,), (1,)), ((), ())),
            preferred_element_type=jnp.float32)
        rid = i * CB + lax.broadcasted_iota(jnp.int32, (CB, 1), 0)
        valid = rid < N
        dist1 = jnp.min(d, axis=1)
        idx1[0, 0, :] = jnp.argmin(d, axis=1).astype(jnp.int32)
        dm = jnp.where(valid, d, jnp.inf)
        colmin = jnp.min(dm, axis=0)[None, :]

        @pl.when(i == 0)
        def _():
            s1[...] = jnp.zeros((1, 1), jnp.float32)
            d2[...] = jnp.full((1, G), jnp.inf, jnp.float32)

        s1[...] += jnp.sum(jnp.where(valid[:, 0], dist1, 0.0)).reshape(1, 1)
        d2[...] = jnp.minimum(d2[...], colmin)

    return pl.pallas_call(
        body,
        grid=(N_PAD // CB,),
        in_specs=[_row_spec(CB, 16), _full_spec((G, 16))],
        out_specs=[_full_spec((1, 1)),
                   pl.BlockSpec((1, 1, CB), lambda i: (i, 0, 0)),
                   _full_spec((1, G))],
        out_shape=[jax.ShapeDtypeStruct((1, 1), jnp.float32),
                   jax.ShapeDtypeStruct((N_PAD // CB, 1, CB), jnp.int32),
                   jax.ShapeDtypeStruct((1, G), jnp.float32)],
        interpret=interpret,
    )


@functools.lru_cache(maxsize=None)
def _tc_lap_eloss(interpret=False):
    def body(cn, sn, cp, sp, din, dout, scal):
        i = pl.program_id(0)
        deg = din[0, :, 0] + din[1, :, 0]
        maxd = jnp.maximum(deg, 1.0)[:, None]
        wsum = deg + dout[0, :, 0] + dout[1, :, 0]
        snew = sn[0] + sn[1]
        sprev = sp[0] + sp[1]
        lap_new = cn[...] - snew / maxd
        lap_old = cp[...] - sprev / maxd
        dd = lap_new - lap_old
        rid = i * CB + lax.broadcasted_iota(jnp.int32, (CB, 1), 0)
        valid = (rid < N)[:, 0]
        lap = jnp.sum(jnp.where(valid, jnp.sum(dd * dd, axis=1), 0.0))
        cn2 = jnp.sum(cn[...] * cn[...], axis=1)
        rdot = jnp.sum(cn[...] * snew, axis=1)
        el = jnp.sum(jnp.where(valid, wsum * cn2 - 2.0 * rdot, 0.0))

        @pl.when(i == 0)
        def _():
            scal[...] = jnp.zeros((1, 2), jnp.float32)

        col = lax.broadcasted_iota(jnp.int32, (1, 2), 1)
        scal[...] += jnp.where(col == 0, lap, el)

    return pl.pallas_call(
        body,
        grid=(N_PAD // CB,),
        in_specs=[_row_spec(CB, 16), _parts_spec(CB, 16),
                  _row_spec(CB, 16), _parts_spec(CB, 16),
                  _parts_spec(CB, 16), _parts_spec(CB, 16)],
        out_specs=_full_spec((1, 2)),
        out_shape=jax.ShapeDtypeStruct((1, 2), jnp.float32),
        interpret=interpret,
    )


@functools.lru_cache(maxsize=None)
def _tc_loss(interpret=False):
    def body(s1a, d2a, sca, npa, s1b, d2b, scb, npb, o):
        closs = (s1a[0, 0] + s1b[0, 0]) / N \
            + jnp.mean(d2a[...]) + jnp.mean(d2b[...])
        sc_sum = sca[...] + scb[...]
        lap = sc_sum[0, 0] / N
        el = sc_sum[0, 1] / E
        nl = (jnp.sum(npa[...]) + jnp.sum(npb[...])) / E
        o[...] = (closs + LAMBDA_N * nl + LAMBDA_LAP * lap
                  + LAMBDA_E * el).reshape(1, 1)

    return pl.pallas_call(
        body,
        grid=(1,),
        in_specs=[_full_spec((1, 1)), _full_spec((1, G)), _full_spec((1, 2)),
                  _full_spec((NW, 16))] * 2,
        out_specs=_full_spec((1, 1)),
        out_shape=jax.ShapeDtypeStruct((1, 1), jnp.float32),
        interpret=interpret,
    )


def kernel(x, c, image_features, gt, gt_normals, edge_index, Pid, params):
    i32 = jnp.int32
    src = edge_index[0].astype(i32)
    dst = edge_index[1].astype(i32)
    order = jnp.argsort(dst, stable=True)
    ssrc = src[order]
    sdst = dst[order]
    src3, dst3 = _edge_layout(ssrc, sdst, SIZES_W128)
    src3b, dst3b = _edge_layout(ssrc, sdst, SIZES_W256)
    x_p = jnp.pad(x, ((0, N_PAD - N), (0, 0)))
    c16 = jnp.pad(c, ((0, N_PAD - N), (0, 16 - DIM)))
    imgT = jnp.transpose(image_features, (1, 2, 0)).reshape(IMH * IMW, CH)
    gt16 = jnp.pad(gt, ((0, 0), (0, 16 - DIM)))
    gtn16 = jnp.pad(gt_normals, ((0, 0), (0, 16 - DIM)))

    eye16 = jnp.eye(16, dtype=jnp.int32)
    _seg = _sc_segsum_exact()
    segsum128 = lambda y, s3, d3: _seg(y, eye16, s3, d3)
    segsum16 = _sc_segsum(16)
    gather128 = _sc_gather(IMH * IMW, CH)
    gather16 = _sc_gather(G, 16)
    nloss_k = _sc_nloss_build()

    dst03, srcD3 = _edge_layout(sdst, ssrc, SIZES_W128)
    e0_table = jnp.zeros((N_PAD, 16), jnp.float32).at[:, 0].set(1.0)
    din = segsum16(e0_table, src3, dst3)
    dout = segsum16(e0_table, dst03, srcD3)
    s_prev_parts = segsum16(c16, src3, dst3)
    pidx = _tc_pidx()(c16)

    h = x_p
    per_layer = []
    for layer in params:
        fetched = gather128(imgT, pidx.reshape(NW, GJ, GCHUNK))
        px_parts = segsum128(h, src3b, dst3b)
        pf_parts = segsum128(fetched, src3b, dst3b)
        h = _tc_gcn_in()(h, px_parts, fetched, pf_parts, din, layer['W_in'],
                         layer['b_in'].reshape(1, FEAT))
        for wd, bd in zip(layer['W_hidden'], layer['b_hidden']):
            p = segsum128(h, src3, dst3)
            h = _tc_gcn_hidden()(h, p, din, wd, bd.reshape(1, FEAT))
        w_out16 = jnp.pad(layer['W_out'], ((0, 0), (0, 16 - DIM)))
        b_out16 = jnp.pad(layer['b_out'], (0, 16 - DIM)).reshape(1, 16)
        p = segsum128(h, src3, dst3)
        c_new16, pidx = _tc_gcn_out()(h, p, din, w_out16, b_out16)
        snew_parts = segsum16(c_new16, src3, dst3)
        scal = _tc_lap_eloss()(c_new16, snew_parts, c16, s_prev_parts, din, dout)
        c16 = c_new16
        s_prev_parts = snew_parts
        s1, idx1, d2 = _tc_chamfer()(c16, gt16)
        nrm16 = gather16(gtn16, idx1.reshape(NW, GJ, GCHUNK))
        npart = nloss_k(c16[:, :4].reshape(-1), nrm16[:, :4].reshape(-1),
                        src3, dst3)
        per_layer.append((s1, d2, scal, npart))

    (s1a, d2a, sca, npa), (s1b, d2b, scb, npb) = per_layer
    loss11 = _tc_loss()(s1a, d2a, sca, npa, s1b, d2b, scb, npb)
    return (h[:N], c16[:N, :DIM], loss11.reshape(()), Pid)
